# Initial kernel scaffold; baseline (speedup 1.0000x reference)
import jax
import jax.numpy as jnp
from jax.experimental import pallas as pl


def kernel(x):
    raise NotImplementedError("write your pallas kernel here")


# speedup vs baseline: 6284.5138x; 6284.5138x over previous
import functools

import jax
import jax.numpy as jnp
from jax import lax
from jax.experimental import pallas as pl

B, D, H, W = 2, 4, 384, 384
BONUS = 10.0


def _pad_hw(p):
    p = jnp.concatenate([p[:, :1], p, p[:, -1:]], axis=1)
    p = jnp.concatenate([p[:1, :], p, p[-1:, :]], axis=0)
    return p


def _kern(x_ref, y_ref, c_ref):
    x = x_ref[0]
    planes = [x[d] for d in range(D)]
    padded = [_pad_hw(p) for p in planes]

    w3 = [jnp.maximum(jnp.maximum(p[:, 0:W], p[:, 1:W + 1]), p[:, 2:W + 2])
          for p in padded]
    m9 = [jnp.maximum(jnp.maximum(a[0:H], a[1:H + 1]), a[2:H + 2])
          for a in w3]
    ring8 = []
    for d in range(D):
        p = padded[d]
        lr = jnp.maximum(p[1:H + 1, 0:W], p[1:H + 1, 2:W + 2])
        ring8.append(jnp.maximum(jnp.maximum(w3[d][0:H], w3[d][2:H + 2]), lr))

    iota_w = lax.broadcasted_iota(jnp.int32, (H, W), 1).astype(jnp.float32)
    iota_h = lax.broadcasted_iota(jnp.int32, (H, W), 0).astype(jnp.float32)

    for d in range(D):
        dm, dp = max(d - 1, 0), min(d + 1, D - 1)
        x0 = planes[d]
        mx = jnp.maximum(jnp.maximum(m9[dm], m9[dp]), ring8[d])
        nms = x0 > mx

        pc, pm, pp = padded[d], padded[dm], padded[dp]
        cc_m = pm[1:H + 1, 1:W + 1]
        cc_p = pp[1:H + 1, 1:W + 1]
        e = pc[1:H + 1, 2:W + 2]
        w_ = pc[1:H + 1, 0:W]
        s_ = pc[2:H + 2, 1:W + 1]
        n_ = pc[0:H, 1:W + 1]

        gx = 0.5 * (e - w_)
        gy = 0.5 * (s_ - n_)
        gs = 0.5 * (cc_p - cc_m)

        dxx = e + w_ - 2.0 * x0
        dyy = s_ + n_ - 2.0 * x0
        dss = cc_p + cc_m - 2.0 * x0
        dxy = 0.25 * (pc[0:H, 0:W] - pc[0:H, 2:W + 2]
                      - pc[2:H + 2, 0:W] + pc[2:H + 2, 2:W + 2])
        dys = 0.25 * (pm[0:H, 1:W + 1] - pm[2:H + 2, 1:W + 1]
                      - pp[0:H, 1:W + 1] + pp[2:H + 2, 1:W + 1])
        dxs = 0.25 * (pm[1:H + 1, 0:W] - pm[1:H + 1, 2:W + 2]
                      - pp[1:H + 1, 0:W] + pp[1:H + 1, 2:W + 2])

        c00 = dyy * dss - dys * dys
        c01 = dxy * dss - dys * dxs
        c02 = dxy * dys - dyy * dxs
        det = dxx * c00 - dxy * c01 + dxs * c02
        ok = (jnp.abs(det) > 0.0) & jnp.isfinite(det)

        inv_det = 1.0 / det
        a01 = -c01
        a11 = dxx * dss - dxs * dxs
        a12 = dxy * dxs - dxx * dys
        a22 = dxx * dyy - dxy * dxy
        x_sol = (c00 * gx + a01 * gy + c02 * gs) * inv_det
        y_sol = (a01 * gx + a11 * gy + a12 * gs) * inv_det
        s_sol = (c02 * gx + a12 * gy + a22 * gs) * inv_det
        finite = (jnp.isfinite(x_sol) & jnp.isfinite(y_sol)
                  & jnp.isfinite(s_sol))
        new_nms = nms & ok & finite

        zero = jnp.zeros_like(x0)
        dx0 = jnp.where(new_nms, -x_sol, zero)
        dx1 = jnp.where(new_nms, -y_sol, zero)
        dx2 = jnp.where(new_nms, -s_sol, zero)
        big = jnp.maximum(jnp.maximum(jnp.abs(dx0), jnp.abs(dx1)),
                          jnp.abs(dx2)) > 0.7
        dx0 = jnp.where(big, zero, dx0)
        dx1 = jnp.where(big, zero, dx1)
        dx2 = jnp.where(big, zero, dx2)

        dy = 0.5 * (gx * dx0 + gy * dx1 + gs * dx2)
        y_ref[0, d] = x0 + dy + BONUS * new_nms.astype(jnp.float32)

        c_ref[0, 0, d] = float(d) + dx2
        c_ref[0, 1, d] = iota_w + dx0
        c_ref[0, 2, d] = iota_h + dx1


@functools.partial(jax.jit, static_argnums=())
def kernel(x):
    xb = x.reshape(B, D, H, W)
    y, coords = pl.pallas_call(
        _kern,
        grid=(B,),
        in_specs=[pl.BlockSpec((1, D, H, W), lambda b: (b, 0, 0, 0))],
        out_specs=[
            pl.BlockSpec((1, D, H, W), lambda b: (b, 0, 0, 0)),
            pl.BlockSpec((1, 3, D, H, W), lambda b: (b, 0, 0, 0, 0)),
        ],
        out_shape=[
            jax.ShapeDtypeStruct((B, D, H, W), jnp.float32),
            jax.ShapeDtypeStruct((B, 3, D, H, W), jnp.float32),
        ],
    )(xb)
    return coords.reshape(B, 1, 3, D, H, W), y.reshape(B, 1, D, H, W)


# speedup vs baseline: 8171.6073x; 1.3003x over previous
import jax
import jax.numpy as jnp
from jax import lax
from jax.experimental import pallas as pl
from jax.experimental.pallas import tpu as pltpu

B, D, H, W = 2, 4, 384, 384
T = 4
HB = H // T
BONUS = 10.0


def _kern(xp_ref, xc_ref, xn_ref, y_ref, c_ref):
    t = pl.program_id(1)
    xc = xc_ref[0]
    first = t == 0
    last = t == T - 1

    planes = [xc[d, 0] for d in range(D)]
    padded = []
    for d in range(D):
        above = jnp.where(first, planes[d][:1], xp_ref[0, d, 0, HB - 1:HB])
        below = jnp.where(last, planes[d][HB - 1:HB], xn_ref[0, d, 0, :1])
        rows = jnp.concatenate([above, planes[d], below], axis=0)
        rows = jnp.concatenate([rows[:, :1], rows, rows[:, -1:]], axis=1)
        padded.append(rows)

    w3 = [jnp.maximum(jnp.maximum(p[:, 0:W], p[:, 1:W + 1]), p[:, 2:W + 2])
          for p in padded]
    m9 = [jnp.maximum(jnp.maximum(a[0:HB], a[1:HB + 1]), a[2:HB + 2])
          for a in w3]
    ring8 = []
    for d in range(D):
        p = padded[d]
        lr = jnp.maximum(p[1:HB + 1, 0:W], p[1:HB + 1, 2:W + 2])
        ring8.append(jnp.maximum(jnp.maximum(w3[d][0:HB], w3[d][2:HB + 2]), lr))

    iota_w = lax.broadcasted_iota(jnp.int32, (HB, W), 1).astype(jnp.float32)
    iota_h = (lax.broadcasted_iota(jnp.int32, (HB, W), 0)
              + t * HB).astype(jnp.float32)

    for d in range(D):
        dm, dp = max(d - 1, 0), min(d + 1, D - 1)
        x0 = planes[d]
        mx = jnp.maximum(jnp.maximum(m9[dm], m9[dp]), ring8[d])
        nms = x0 > mx

        pc, pm, pp = padded[d], padded[dm], padded[dp]
        cc_m = pm[1:HB + 1, 1:W + 1]
        cc_p = pp[1:HB + 1, 1:W + 1]
        e = pc[1:HB + 1, 2:W + 2]
        w_ = pc[1:HB + 1, 0:W]
        s_ = pc[2:HB + 2, 1:W + 1]
        n_ = pc[0:HB, 1:W + 1]

        gx = 0.5 * (e - w_)
        gy = 0.5 * (s_ - n_)
        gs = 0.5 * (cc_p - cc_m)

        dxx = e + w_ - 2.0 * x0
        dyy = s_ + n_ - 2.0 * x0
        dss = cc_p + cc_m - 2.0 * x0
        dxy = 0.25 * (pc[0:HB, 0:W] - pc[0:HB, 2:W + 2]
                      - pc[2:HB + 2, 0:W] + pc[2:HB + 2, 2:W + 2])
        dys = 0.25 * (pm[0:HB, 1:W + 1] - pm[2:HB + 2, 1:W + 1]
                      - pp[0:HB, 1:W + 1] + pp[2:HB + 2, 1:W + 1])
        dxs = 0.25 * (pm[1:HB + 1, 0:W] - pm[1:HB + 1, 2:W + 2]
                      - pp[1:HB + 1, 0:W] + pp[1:HB + 1, 2:W + 2])

        c00 = dyy * dss - dys * dys
        c01 = dxy * dss - dys * dxs
        c02 = dxy * dys - dyy * dxs
        det = dxx * c00 - dxy * c01 + dxs * c02
        ok = (jnp.abs(det) > 0.0) & jnp.isfinite(det)

        inv_det = 1.0 / det
        a01 = -c01
        a11 = dxx * dss - dxs * dxs
        a12 = dxy * dxs - dxx * dys
        a22 = dxx * dyy - dxy * dxy
        x_sol = (c00 * gx + a01 * gy + c02 * gs) * inv_det
        y_sol = (a01 * gx + a11 * gy + a12 * gs) * inv_det
        s_sol = (c02 * gx + a12 * gy + a22 * gs) * inv_det
        finite = (jnp.isfinite(x_sol) & jnp.isfinite(y_sol)
                  & jnp.isfinite(s_sol))
        new_nms = nms & ok & finite

        zero = jnp.zeros_like(x0)
        dx0 = jnp.where(new_nms, -x_sol, zero)
        dx1 = jnp.where(new_nms, -y_sol, zero)
        dx2 = jnp.where(new_nms, -s_sol, zero)
        big = jnp.maximum(jnp.maximum(jnp.abs(dx0), jnp.abs(dx1)),
                          jnp.abs(dx2)) > 0.7
        dx0 = jnp.where(big, zero, dx0)
        dx1 = jnp.where(big, zero, dx1)
        dx2 = jnp.where(big, zero, dx2)

        dy = 0.5 * (gx * dx0 + gy * dx1 + gs * dx2)
        y_ref[0, d, 0] = x0 + dy + BONUS * new_nms.astype(jnp.float32)

        c_ref[0, 0, d, 0] = float(d) + dx2
        c_ref[0, 1, d, 0] = iota_w + dx0
        c_ref[0, 2, d, 0] = iota_h + dx1


def kernel(x):
    xt = x.reshape(B, D, T, HB, W)

    def mk_spec(off):
        return pl.BlockSpec(
            (1, D, 1, HB, W),
            lambda b, t: (b, 0, jnp.clip(t + off, 0, T - 1), 0, 0))

    y, coords = pl.pallas_call(
        _kern,
        grid=(B, T),
        in_specs=[mk_spec(-1), mk_spec(0), mk_spec(1)],
        out_specs=[
            pl.BlockSpec((1, D, 1, HB, W), lambda b, t: (b, 0, t, 0, 0)),
            pl.BlockSpec((1, 3, D, 1, HB, W),
                         lambda b, t: (b, 0, 0, t, 0, 0)),
        ],
        out_shape=[
            jax.ShapeDtypeStruct((B, D, T, HB, W), jnp.float32),
            jax.ShapeDtypeStruct((B, 3, D, T, HB, W), jnp.float32),
        ],
        compiler_params=pltpu.CompilerParams(
            dimension_semantics=("parallel", "parallel")),
    )(xt, xt, xt)
    coords = coords.reshape(B, 1, 3, D, H, W)
    y = y.reshape(B, 1, D, H, W)
    return coords, y


# speedup vs baseline: 18110.3600x; 2.2163x over previous
import jax
import jax.numpy as jnp
from jax import lax
from jax.experimental import pallas as pl
from jax.experimental.pallas import tpu as pltpu

B, D, H, W = 2, 4, 384, 384
T = 4
HB = H // T
BONUS = 10.0


def _kern(xp_ref, xc_ref, xn_ref, y_ref, c_ref):
    t = pl.program_id(1)
    first = t == 0
    last = t == T - 1

    planes, rows, a_w, b_e, emw, w3r = [], [], [], [], [], []
    m9, ring8 = [], []
    for d in range(D):
        x0 = xc_ref[0, d, 0]
        planes.append(x0)
        above = jnp.where(first, x0[:1], xp_ref[0, d, 0, HB - 1:HB])
        below = jnp.where(last, x0[HB - 1:HB], xn_ref[0, d, 0, :1])
        r = jnp.concatenate([above, x0, below], axis=0)
        rows.append(r)
        a = jnp.concatenate([r[:, :1], r[:, :W - 1]], axis=1)
        b = jnp.concatenate([r[:, 1:], r[:, W - 1:]], axis=1)
        a_w.append(a)
        b_e.append(b)
        emw.append(b - a)
        w3r.append(jnp.maximum(jnp.maximum(a, r), b))

    for d in range(D):
        w3 = w3r[d]
        m9.append(jnp.maximum(jnp.maximum(w3[0:HB], w3[1:HB + 1]),
                              w3[2:HB + 2]))
        lr = jnp.maximum(a_w[d][1:HB + 1], b_e[d][1:HB + 1])
        ring8.append(jnp.maximum(jnp.maximum(w3[0:HB], w3[2:HB + 2]), lr))

    iota_w = lax.broadcasted_iota(jnp.int32, (HB, W), 1).astype(jnp.float32)
    iota_h = (lax.broadcasted_iota(jnp.int32, (HB, W), 0)
              + t * HB).astype(jnp.float32)

    for d in range(D):
        dm, dp = max(d - 1, 0), min(d + 1, D - 1)
        x0 = planes[d]
        mx = jnp.maximum(jnp.maximum(m9[dm], m9[dp]), ring8[d])
        nms = x0 > mx

        r = rows[d]
        n_c = r[0:HB]
        s_c = r[2:HB + 2]
        emw_c = emw[d][1:HB + 1]

        gx = 0.5 * emw_c
        gy = 0.5 * (s_c - n_c)
        gs = 0.5 * (planes[dp] - planes[dm])

        dxx = a_w[d][1:HB + 1] + b_e[d][1:HB + 1] - 2.0 * x0
        dyy = s_c + n_c - 2.0 * x0
        dss = planes[dp] + planes[dm] - 2.0 * x0
        dxy = 0.25 * (emw[d][2:HB + 2] - emw[d][0:HB])
        dys = 0.25 * (rows[dm][0:HB] - rows[dm][2:HB + 2]
                      - rows[dp][0:HB] + rows[dp][2:HB + 2])
        dxs = 0.25 * (emw[dp][1:HB + 1] - emw[dm][1:HB + 1])

        c00 = dyy * dss - dys * dys
        c01 = dxy * dss - dys * dxs
        c02 = dxy * dys - dyy * dxs
        det = dxx * c00 - dxy * c01 + dxs * c02
        ok = (jnp.abs(det) > 0.0) & jnp.isfinite(det)

        inv_det = 1.0 / det
        a01 = -c01
        a11 = dxx * dss - dxs * dxs
        a12 = dxy * dxs - dxx * dys
        a22 = dxx * dyy - dxy * dxy
        x_sol = (c00 * gx + a01 * gy + c02 * gs) * inv_det
        y_sol = (a01 * gx + a11 * gy + a12 * gs) * inv_det
        s_sol = (c02 * gx + a12 * gy + a22 * gs) * inv_det
        finite = (jnp.isfinite(x_sol) & jnp.isfinite(y_sol)
                  & jnp.isfinite(s_sol))
        new_nms = nms & ok & finite

        amax = jnp.maximum(jnp.maximum(jnp.abs(x_sol), jnp.abs(y_sol)),
                           jnp.abs(s_sol))
        keep = new_nms & (amax <= 0.7)

        zero = jnp.zeros_like(x0)
        dx0 = jnp.where(keep, -x_sol, zero)
        dx1 = jnp.where(keep, -y_sol, zero)
        dx2 = jnp.where(keep, -s_sol, zero)

        dy = 0.5 * (gx * dx0 + gy * dx1 + gs * dx2)
        y_ref[0, d, 0] = x0 + dy + BONUS * new_nms.astype(jnp.float32)

        c_ref[0, 0, d, 0] = float(d) + dx2
        c_ref[0, 1, d, 0] = iota_w + dx0
        c_ref[0, 2, d, 0] = iota_h + dx1


def kernel(x):
    xt = x.reshape(B, D, T, HB, W)

    def mk_spec(off):
        return pl.BlockSpec(
            (1, D, 1, HB, W),
            lambda b, t: (b, 0, jnp.clip(t + off, 0, T - 1), 0, 0))

    y, coords = pl.pallas_call(
        _kern,
        grid=(B, T),
        in_specs=[mk_spec(-1), mk_spec(0), mk_spec(1)],
        out_specs=[
            pl.BlockSpec((1, D, 1, HB, W), lambda b, t: (b, 0, t, 0, 0)),
            pl.BlockSpec((1, 3, D, 1, HB, W),
                         lambda b, t: (b, 0, 0, t, 0, 0)),
        ],
        out_shape=[
            jax.ShapeDtypeStruct((B, D, T, HB, W), jnp.float32),
            jax.ShapeDtypeStruct((B, 3, D, T, HB, W), jnp.float32),
        ],
        compiler_params=pltpu.CompilerParams(
            dimension_semantics=("parallel", "parallel")),
    )(xt, xt, xt)
    coords = coords.reshape(B, 1, 3, D, H, W)
    y = y.reshape(B, 1, D, H, W)
    return coords, y


# speedup vs baseline: 18723.2690x; 1.0338x over previous
import jax
import jax.numpy as jnp
from jax import lax
from jax.experimental import pallas as pl
from jax.experimental.pallas import tpu as pltpu

B, D, H, W = 2, 4, 384, 384
T = 4
HB = H // T
PR = HB + 16
BONUS = 10.0


def _kern(xp_ref, xc_ref, xn_ref, y_ref, c_ref, xs_ref, emw_ref, w3_ref):
    t = pl.program_id(1)
    first = t == 0
    last = t == T - 1

    planes, a_c, b_c = [], [], []
    for d in range(D):
        x0 = xc_ref[0, d, 0]
        planes.append(x0)
        above = jnp.where(first, x0[:1], xp_ref[0, d, 0, HB - 1:HB])
        below = jnp.where(last, x0[HB - 1:HB], xn_ref[0, d, 0, :1])
        xs_ref[d, 8:8 + HB] = x0
        xs_ref[d, 7:8] = above
        xs_ref[d, 8 + HB:9 + HB] = below

    for d in range(D):
        xv = xs_ref[d]
        a = jnp.concatenate([xv[:, :1], xv[:, :W - 1]], axis=1)
        b = jnp.concatenate([xv[:, 1:], xv[:, W - 1:]], axis=1)
        emw_ref[d] = b - a
        w3_ref[d] = jnp.maximum(jnp.maximum(a, xv), b)
        a_c.append(a[8:8 + HB])
        b_c.append(b[8:8 + HB])

    m9, ring8 = [], []
    for d in range(D):
        w3n = w3_ref[d, 7:7 + HB]
        w3c = w3_ref[d, 8:8 + HB]
        w3s = w3_ref[d, 9:9 + HB]
        m9.append(jnp.maximum(jnp.maximum(w3n, w3c), w3s))
        lr = jnp.maximum(a_c[d], b_c[d])
        ring8.append(jnp.maximum(jnp.maximum(w3n, w3s), lr))

    iota_w = lax.broadcasted_iota(jnp.int32, (HB, W), 1).astype(jnp.float32)
    iota_h = (lax.broadcasted_iota(jnp.int32, (HB, W), 0)
              + t * HB).astype(jnp.float32)

    for d in range(D):
        dm, dp = max(d - 1, 0), min(d + 1, D - 1)
        x0 = planes[d]
        mx = jnp.maximum(jnp.maximum(m9[dm], m9[dp]), ring8[d])
        nms = x0 > mx

        n_c = xs_ref[d, 7:7 + HB]
        s_c = xs_ref[d, 9:9 + HB]
        emw_c = emw_ref[d, 8:8 + HB]

        gx = 0.5 * emw_c
        gy = 0.5 * (s_c - n_c)
        gs = 0.5 * (planes[dp] - planes[dm])

        dxx = a_c[d] + b_c[d] - 2.0 * x0
        dyy = s_c + n_c - 2.0 * x0
        dss = planes[dp] + planes[dm] - 2.0 * x0
        dxy = 0.25 * (emw_ref[d, 9:9 + HB] - emw_ref[d, 7:7 + HB])
        dys = 0.25 * (xs_ref[dm, 7:7 + HB] - xs_ref[dm, 9:9 + HB]
                      - xs_ref[dp, 7:7 + HB] + xs_ref[dp, 9:9 + HB])
        dxs = 0.25 * (emw_ref[dp, 8:8 + HB] - emw_ref[dm, 8:8 + HB])

        c00 = dyy * dss - dys * dys
        c01 = dxy * dss - dys * dxs
        c02 = dxy * dys - dyy * dxs
        det = dxx * c00 - dxy * c01 + dxs * c02
        ok = (jnp.abs(det) > 0.0) & jnp.isfinite(det)

        inv_det = 1.0 / det
        a01 = -c01
        a11 = dxx * dss - dxs * dxs
        a12 = dxy * dxs - dxx * dys
        a22 = dxx * dyy - dxy * dxy
        x_sol = (c00 * gx + a01 * gy + c02 * gs) * inv_det
        y_sol = (a01 * gx + a11 * gy + a12 * gs) * inv_det
        s_sol = (c02 * gx + a12 * gy + a22 * gs) * inv_det
        finite = (jnp.isfinite(x_sol) & jnp.isfinite(y_sol)
                  & jnp.isfinite(s_sol))
        new_nms = nms & ok & finite

        amax = jnp.maximum(jnp.maximum(jnp.abs(x_sol), jnp.abs(y_sol)),
                           jnp.abs(s_sol))
        keep = new_nms & (amax <= 0.7)

        zero = jnp.zeros_like(x0)
        dx0 = jnp.where(keep, -x_sol, zero)
        dx1 = jnp.where(keep, -y_sol, zero)
        dx2 = jnp.where(keep, -s_sol, zero)

        dy = 0.5 * (gx * dx0 + gy * dx1 + gs * dx2)
        y_ref[0, d, 0] = x0 + dy + BONUS * new_nms.astype(jnp.float32)

        c_ref[0, 0, d, 0] = float(d) + dx2
        c_ref[0, 1, d, 0] = iota_w + dx0
        c_ref[0, 2, d, 0] = iota_h + dx1


def kernel(x):
    xt = x.reshape(B, D, T, HB, W)

    def mk_spec(off):
        return pl.BlockSpec(
            (1, D, 1, HB, W),
            lambda b, t: (b, 0, jnp.clip(t + off, 0, T - 1), 0, 0))

    y, coords = pl.pallas_call(
        _kern,
        grid=(B, T),
        in_specs=[mk_spec(-1), mk_spec(0), mk_spec(1)],
        out_specs=[
            pl.BlockSpec((1, D, 1, HB, W), lambda b, t: (b, 0, t, 0, 0)),
            pl.BlockSpec((1, 3, D, 1, HB, W),
                         lambda b, t: (b, 0, 0, t, 0, 0)),
        ],
        out_shape=[
            jax.ShapeDtypeStruct((B, D, T, HB, W), jnp.float32),
            jax.ShapeDtypeStruct((B, 3, D, T, HB, W), jnp.float32),
        ],
        scratch_shapes=[
            pltpu.VMEM((D, PR, W), jnp.float32),
            pltpu.VMEM((D, PR, W), jnp.float32),
            pltpu.VMEM((D, PR, W), jnp.float32),
        ],
        compiler_params=pltpu.CompilerParams(
            dimension_semantics=("parallel", "parallel")),
    )(xt, xt, xt)
    coords = coords.reshape(B, 1, 3, D, H, W)
    y = y.reshape(B, 1, D, H, W)
    return coords, y
